# Initial kernel scaffold; baseline (speedup 1.0000x reference)
import jax
import jax.numpy as jnp
from jax.experimental import pallas as pl


def kernel(x, edge_index, edge_weight, W1, b1, W2, b2):
    raise NotImplementedError("write your pallas kernel here")


# speedup vs baseline: 3.6646x; 3.6646x over previous
import functools

import jax
import jax.numpy as jnp
from jax import lax
from jax.experimental import pallas as pl
from jax.experimental.pallas import tpu as pltpu
from jax.experimental.pallas import tpu_sc as plsc

N = 10000
NFEAT = 128
NHID = 128
NCLASS = 40
NCPAD = 64

CHUNK = 128

_BCAST_DNUMS = lax.GatherDimensionNumbers(
    offset_dims=(), collapsed_slice_dims=(0,), start_index_map=(0,))


def _bcast_lane(vec, lane):
  idx = jnp.full((16, 1), lane, jnp.int32)
  return lax.gather(vec, idx, _BCAST_DNUMS, (1,),
                    mode=lax.GatherScatterMode.PROMISE_IN_BOUNDS)
NCORES = 2
NSUB = 16
NWORKERS = NCORES * NSUB
N_PAD = 10240
ROWS_PER_TILE = N_PAD // NSUB
ZCHUNK = 128


def _make_spmm(D, chunks_per_worker):
  epw = chunks_per_worker * CHUNK
  mesh = plsc.VectorSubcoreMesh(core_axis_name="c", subcore_axis_name="s")

  @functools.partial(
      pl.kernel,
      mesh=mesh,
      out_type=jax.ShapeDtypeStruct((NCORES, N_PAD, D), jnp.float32),
      scratch_types=[
          pltpu.VMEM((CHUNK,), jnp.int32),
          pltpu.VMEM((CHUNK,), jnp.int32),
          pltpu.VMEM((CHUNK,), jnp.float32),
          pltpu.VMEM((CHUNK, D), jnp.float32),
          pltpu.VMEM_SHARED((N_PAD, D), jnp.float32),
          pltpu.SemaphoreType.DMA,
      ],
      compiler_params=pltpu.CompilerParams(use_tc_tiling_on_sc=False),
  )
  def spmm(sup_hbm, src_hbm, dst_hbm, w_hbm, out_hbm,
           src_v, dst_v, w_v, rows_v, acc, sem):
    cid = lax.axis_index("c")
    sid = lax.axis_index("s")
    wid = sid * NCORES + cid

    def zrow(r, carry):
      for f in range(D // 16):
        rows_v[r, pl.ds(f * 16, 16)] = jnp.zeros((16,), jnp.float32)
      return carry
    lax.fori_loop(0, ZCHUNK, zrow, 0)
    for i in range(ROWS_PER_TILE // ZCHUNK):
      pltpu.sync_copy(
          rows_v.at[pl.ds(0, ZCHUNK)],
          acc.at[pl.ds(
              pl.multiple_of(sid * ROWS_PER_TILE + i * ZCHUNK, ZCHUNK),
              ZCHUNK)])
    plsc.subcore_barrier()

    def step(c, carry):
      base = pl.multiple_of(wid * epw + c * CHUNK, CHUNK)
      pltpu.sync_copy(src_hbm.at[pl.ds(base, CHUNK)], src_v)
      pltpu.sync_copy(dst_hbm.at[pl.ds(base, CHUNK)], dst_v)
      pltpu.sync_copy(w_hbm.at[pl.ds(base, CHUNK)], w_v)
      pltpu.async_copy(sup_hbm.at[src_v], rows_v, sem).wait()

      def scale(g, carry2):
        wv = w_v[pl.ds(pl.multiple_of(g * 16, 16), 16)]
        for e in range(16):
          wb = _bcast_lane(wv, e)
          r = g * 16 + e
          for f in range(D // 16):
            rows_v[r, pl.ds(f * 16, 16)] = rows_v[r, pl.ds(f * 16, 16)] * wb
        return carry2
      lax.fori_loop(0, CHUNK // 16, scale, 0)

      pltpu.sync_copy(rows_v, acc.at[dst_v], add=True)
      return carry
    lax.fori_loop(0, chunks_per_worker, step, 0)

    plsc.subcore_barrier()
    row0 = pl.multiple_of(sid * ROWS_PER_TILE, ZCHUNK)
    pltpu.sync_copy(acc.at[pl.ds(row0, ROWS_PER_TILE)],
                    out_hbm.at[cid].at[pl.ds(row0, ROWS_PER_TILE)])

  return spmm


def _mm1(x, w1):
  bm = 1000

  def body(x_ref, w_ref, o_ref):
    o_ref[...] = jnp.dot(x_ref[...], w_ref[...],
                         preferred_element_type=jnp.float32)

  return pl.pallas_call(
      body,
      grid=(N // bm,),
      in_specs=[
          pl.BlockSpec((bm, NFEAT), lambda i: (i, 0)),
          pl.BlockSpec((NFEAT, NHID), lambda i: (0, 0)),
      ],
      out_specs=pl.BlockSpec((bm, NHID), lambda i: (i, 0)),
      out_shape=jax.ShapeDtypeStruct((N, NHID), jnp.float32),
  )(x, w1)


def _mm2_fused(parts, b1, w2p):
  bm = 1000

  def body(p_ref, b_ref, w_ref, o_ref):
    h = jnp.maximum(p_ref[0] + p_ref[1] + b_ref[...], 0.0)
    o_ref[...] = jnp.dot(h, w_ref[...], preferred_element_type=jnp.float32)

  return pl.pallas_call(
      body,
      grid=(N // bm,),
      in_specs=[
          pl.BlockSpec((2, bm, NHID), lambda i: (0, i, 0)),
          pl.BlockSpec((1, NHID), lambda i: (0, 0)),
          pl.BlockSpec((NHID, NCPAD), lambda i: (0, 0)),
      ],
      out_specs=pl.BlockSpec((bm, NCPAD), lambda i: (i, 0)),
      out_shape=jax.ShapeDtypeStruct((N, NCPAD), jnp.float32),
  )(parts, b1.reshape(1, NHID), w2p)


def _log_softmax(parts2, b2p):
  bm = 1000

  def body(q_ref, b_ref, o_ref):
    zb = q_ref[0] + q_ref[1] + b_ref[...]
    col = lax.broadcasted_iota(jnp.int32, (bm, NCPAD), 1)
    valid = col < NCLASS
    logits = jnp.where(valid, zb, -jnp.inf)
    m = jnp.max(logits, axis=1, keepdims=True)
    e = jnp.where(valid, jnp.exp(zb - m), 0.0)
    lse = jnp.log(jnp.sum(e, axis=1, keepdims=True)) + m
    o_ref[...] = zb - lse

  return pl.pallas_call(
      body,
      grid=(N // bm,),
      in_specs=[
          pl.BlockSpec((2, bm, NCPAD), lambda i: (0, i, 0)),
          pl.BlockSpec((1, NCPAD), lambda i: (0, 0)),
      ],
      out_specs=pl.BlockSpec((bm, NCPAD), lambda i: (i, 0)),
      out_shape=jax.ShapeDtypeStruct((N, NCPAD), jnp.float32),
  )(parts2, b2p)


@jax.jit
def kernel(x, edge_index, edge_weight, W1, b1, W2, b2):
  e = edge_weight.shape[0]
  step = NWORKERS * CHUNK
  e_pad = ((e + step - 1) // step) * step
  cpw = e_pad // step
  pad = e_pad - e

  src = jnp.concatenate([edge_index[0], jnp.zeros((pad,), jnp.int32)])
  dst = jnp.concatenate([edge_index[1], jnp.zeros((pad,), jnp.int32)])
  w = jnp.concatenate([edge_weight, jnp.zeros((pad,), jnp.float32)])

  support1 = _mm1(x, W1)
  parts1 = _make_spmm(NHID, cpw)(support1, src, dst, w)
  w2p = jnp.pad(W2, ((0, 0), (0, NCPAD - NCLASS)))
  support2 = _mm2_fused(parts1, b1, w2p)
  parts2 = _make_spmm(NCPAD, cpw)(support2, src, dst, w)
  b2p = jnp.pad(b2, (0, NCPAD - NCLASS)).reshape(1, NCPAD)
  out = _log_softmax(parts2, b2p)
  return out[:, :NCLASS]
